# baseline (speedup 1.0000x reference)
import functools

import jax
import jax.numpy as jnp
from jax import lax
from jax.experimental import pallas as pl
from jax.experimental.pallas import tpu as pltpu
from jax.experimental.pallas import tpu_sc as plsc

IN_F = 768
OUT_F = 768
R = 8
E = 64
TOPK = 8
NGRAM = 2
GAMMA = 0.5
TEMP = 1.0
ALPHA = 16.0

AB = 1024
TB = 1024

NW = 32
NC = 2
LANES = 16


def _anchor_kernel(x3_ref, la_ref, rh_ref, rd_ref, h_ref, ds_ref, hs_ref, dss_ref):
    j = pl.program_id(0)
    xa = x3_ref[...]
    h = lax.dot_general(xa, rh_ref[...], (((1,), (0,)), ((), ())),
                        preferred_element_type=jnp.float32)
    da = lax.dot_general(xa, la_ref[...], (((1,), (1,)), ((), ())),
                         preferred_element_type=jnp.float32)
    ds = lax.dot_general(da, rd_ref[...], (((1,), (0,)), ((), ())),
                         preferred_element_type=jnp.float32)
    h_ref[...] = h
    ds_ref[...] = ds
    hmax = jnp.max(jnp.abs(h), keepdims=True).reshape(1, 1)
    dmax = jnp.max(jnp.abs(ds), keepdims=True).reshape(1, 1)

    @pl.when(j == 0)
    def _():
        hs_ref[...] = hmax
        dss_ref[...] = dmax

    @pl.when(j > 0)
    def _():
        hs_ref[...] = jnp.maximum(hs_ref[...], hmax)
        dss_ref[...] = jnp.maximum(dss_ref[...], dmax)


def _sc_routing(h_hbm, d_hbm, c_hbm, w_hbm, hbuf, dbuf, wbuf, cbuf, lbuf):
    wid = lax.axis_index("s") * NC + lax.axis_index("c")
    apw = (2 * 8192 // NGRAM) // NW
    flat = apw * E
    base = wid * flat
    pltpu.sync_copy(h_hbm.at[pl.ds(base, flat)], hbuf)
    pltpu.sync_copy(d_hbm.at[pl.ds(base, flat)], dbuf)
    pltpu.sync_copy(c_hbm, cbuf)
    c1 = cbuf[0:LANES]
    c2 = cbuf[LANES:2 * LANES]
    lane64 = lax.broadcasted_iota(jnp.int32, (LANES,), 0) * E

    def group(g, carry):
        goff = g * (LANES * E)
        m = jnp.full((LANES,), -jnp.inf, dtype=jnp.float32)
        for e in range(E):
            idx = lane64 + (goff + e)
            lv = c1 * plsc.load_gather(hbuf, [idx]) + c2 * plsc.load_gather(dbuf, [idx])
            lbuf[e] = lv
            m = jnp.maximum(m, lv)
        for e in range(E):
            lbuf[e] = jnp.exp(lbuf[e] - m)
        mx = jnp.full((LANES,), 1.0, dtype=jnp.float32)
        for _ in range(TOPK):
            nmx = jnp.full((LANES,), 0.0, dtype=jnp.float32)
            for e in range(E):
                a = lbuf[e]
                a = jnp.where(a == mx, -a, a)
                lbuf[e] = a
                nmx = jnp.maximum(nmx, a)
            mx = nmx
        for e in range(E):
            a = lbuf[e]
            w = jnp.where(a < 0.0, -a, jnp.full((LANES,), 0.0, dtype=jnp.float32))
            plsc.store_scatter(wbuf, [lane64 + (goff + e)], w)
        return carry

    lax.fori_loop(0, apw // LANES, group, 0)
    pltpu.sync_copy(wbuf, w_hbm.at[pl.ds(base, flat)])


def _out_kernel(xf_ref, w_ref, la_ref, lb_ref, wm_ref, limes_ref, o_ref):
    xb = xf_ref[...]
    xw = lax.dot_general(xb.astype(jnp.bfloat16), w_ref[...].astype(jnp.bfloat16),
                         (((1,), (1,)), ((), ())),
                         preferred_element_type=jnp.float32)
    delta = lax.dot_general(xb, la_ref[...], (((1,), (1,)), ((), ())),
                            preferred_element_type=jnp.float32)
    wm = wm_ref[...]
    s = jnp.maximum(jnp.sum(wm, axis=-1, keepdims=True), 1e-9)
    mix = lax.dot_general(wm, limes_ref[...], (((1,), (0,)), ((), ())),
                          preferred_element_type=jnp.float32)
    p_mix = mix / s
    hbc = TB // NGRAM
    rows = lax.broadcasted_iota(jnp.int32, (TB, hbc), 0) // NGRAM
    cols = lax.broadcasted_iota(jnp.int32, (TB, hbc), 1)
    erep = (rows == cols).astype(jnp.float32)
    p_full = lax.dot_general(erep, p_mix, (((1,), (0,)), ((), ())),
                             preferred_element_type=jnp.float32)
    q = delta * p_full
    lora = lax.dot_general(q, lb_ref[...], (((1,), (1,)), ((), ())),
                           preferred_element_type=jnp.float32)
    o_ref[...] = xw + lora * (ALPHA / R)


def kernel(x, weight, lora_A, lora_B, router_h, router_d, limes):
    Bsz, T, _ = x.shape
    na = (T // NGRAM) * Bsz
    bt = Bsz * T
    x3 = x.reshape(na, NGRAM * IN_F)
    xf = x.reshape(bt, IN_F)

    h_all, ds_all, hs, dss = pl.pallas_call(
        _anchor_kernel,
        grid=(na // AB,),
        in_specs=[
            pl.BlockSpec((AB, IN_F), lambda j: (j, NGRAM - 1)),
            pl.BlockSpec((R, IN_F), lambda j: (0, 0)),
            pl.BlockSpec((IN_F, E), lambda j: (0, 0)),
            pl.BlockSpec((R, E), lambda j: (0, 0)),
        ],
        out_specs=[
            pl.BlockSpec((AB, E), lambda j: (j, 0)),
            pl.BlockSpec((AB, E), lambda j: (j, 0)),
            pl.BlockSpec((1, 1), lambda j: (0, 0)),
            pl.BlockSpec((1, 1), lambda j: (0, 0)),
        ],
        out_shape=[
            jax.ShapeDtypeStruct((na, E), jnp.float32),
            jax.ShapeDtypeStruct((na, E), jnp.float32),
            jax.ShapeDtypeStruct((1, 1), jnp.float32),
            jax.ShapeDtypeStruct((1, 1), jnp.float32),
        ],
    )(x3, lora_A, router_h, router_d)

    eps = 1e-6
    c1 = (1.0 - GAMMA) / (jnp.maximum(hs[0, 0], eps) * max(TEMP, eps))
    c2 = GAMMA / (jnp.maximum(dss[0, 0], eps) * max(TEMP, eps))
    cvec = jnp.concatenate([jnp.full((LANES,), 1.0) * c1,
                            jnp.full((LANES,), 1.0) * c2]).astype(jnp.float32)

    apw = na // NW
    sc_route = functools.partial(
        pl.kernel,
        out_type=jax.ShapeDtypeStruct((na * E,), jnp.float32),
        mesh=plsc.VectorSubcoreMesh(core_axis_name="c", subcore_axis_name="s"),
        compiler_params=pltpu.CompilerParams(needs_layout_passes=False),
        scratch_types=[
            pltpu.VMEM((apw * E,), jnp.float32),
            pltpu.VMEM((apw * E,), jnp.float32),
            pltpu.VMEM((apw * E,), jnp.float32),
            pltpu.VMEM((2 * LANES,), jnp.float32),
            pltpu.VMEM((E, LANES), jnp.float32),
        ],
    )(_sc_routing)
    w_all = sc_route(h_all.reshape(-1), ds_all.reshape(-1), cvec).reshape(na, E)

    out = pl.pallas_call(
        _out_kernel,
        grid=(bt // TB,),
        in_specs=[
            pl.BlockSpec((TB, IN_F), lambda j: (j, 0)),
            pl.BlockSpec((OUT_F, IN_F), lambda j: (0, 0)),
            pl.BlockSpec((R, IN_F), lambda j: (0, 0)),
            pl.BlockSpec((OUT_F, R), lambda j: (0, 0)),
            pl.BlockSpec((TB // NGRAM, E), lambda j: (j, 0)),
            pl.BlockSpec((E, R), lambda j: (0, 0)),
        ],
        out_specs=pl.BlockSpec((TB, OUT_F), lambda j: (j, 0)),
        out_shape=jax.ShapeDtypeStruct((bt, OUT_F), jnp.float32),
    )(xf, weight, lora_A, lora_B, w_all, limes)

    return out.reshape(Bsz, T, OUT_F)


# speedup vs baseline: 1.0841x; 1.0841x over previous
import functools

import jax
import jax.numpy as jnp
from jax import lax
from jax.experimental import pallas as pl
from jax.experimental.pallas import tpu as pltpu
from jax.experimental.pallas import tpu_sc as plsc

IN_F = 768
OUT_F = 768
R = 8
E = 64
TOPK = 8
NGRAM = 2
GAMMA = 0.5
TEMP = 1.0
ALPHA = 16.0

AB = 1024
TB = 1024

NW = 32
NC = 2
LANES = 16


def _anchor_kernel(x3_ref, la_ref, rh_ref, rd_ref, h_ref, ds_ref, hs_ref, dss_ref):
    j = pl.program_id(0)
    xa = x3_ref[...]
    h = lax.dot_general(xa, rh_ref[...], (((1,), (0,)), ((), ())),
                        preferred_element_type=jnp.float32)
    da = lax.dot_general(xa, la_ref[...], (((1,), (1,)), ((), ())),
                         preferred_element_type=jnp.float32)
    ds = lax.dot_general(da, rd_ref[...], (((1,), (0,)), ((), ())),
                         preferred_element_type=jnp.float32)
    h_ref[...] = h
    ds_ref[...] = ds
    hmax = jnp.max(jnp.abs(h), keepdims=True).reshape(1, 1)
    dmax = jnp.max(jnp.abs(ds), keepdims=True).reshape(1, 1)

    @pl.when(j == 0)
    def _():
        hs_ref[...] = hmax
        dss_ref[...] = dmax

    @pl.when(j > 0)
    def _():
        hs_ref[...] = jnp.maximum(hs_ref[...], hmax)
        dss_ref[...] = jnp.maximum(dss_ref[...], dmax)


def _sc_routing(h_hbm, d_hbm, c_hbm, w_hbm, hbuf, dbuf, wbuf, cbuf, lbuf):
    wid = lax.axis_index("s") * NC + lax.axis_index("c")
    apw = (2 * 8192 // NGRAM) // NW
    flat = apw * E
    base = wid * flat
    pltpu.sync_copy(h_hbm.at[pl.ds(base, flat)], hbuf)
    pltpu.sync_copy(d_hbm.at[pl.ds(base, flat)], dbuf)
    pltpu.sync_copy(c_hbm, cbuf)
    c1 = cbuf[0:LANES]
    c2 = cbuf[LANES:2 * LANES]
    lane64 = lax.broadcasted_iota(jnp.int32, (LANES,), 0) * E

    def tree_max(vals):
        while len(vals) > 1:
            vals = [jnp.maximum(vals[2 * i], vals[2 * i + 1])
                    for i in range(len(vals) // 2)] + vals[len(vals) & ~1:]
        return vals[0]

    zero = jnp.full((LANES,), 0.0, dtype=jnp.float32)

    def group(g, carry):
        goff = g * (LANES * E)
        chunk_ms = []
        for c in range(4):
            chunk = []
            for i in range(LANES):
                e = c * LANES + i
                idx = lane64 + (goff + e)
                lv = (c1 * plsc.load_gather(hbuf, [idx])
                      + c2 * plsc.load_gather(dbuf, [idx]))
                lbuf[e] = lv
                chunk.append(lv)
            chunk_ms.append(tree_max(chunk))
        m = tree_max(chunk_ms)
        for e in range(E):
            lbuf[e] = jnp.exp(lbuf[e] - m)
        mk = jnp.full((LANES,), 1.0, dtype=jnp.float32)
        for _ in range(TOPK - 1):
            cand = []
            for c in range(4):
                chunk = []
                for i in range(LANES):
                    a = lbuf[c * LANES + i]
                    chunk.append(jnp.where(a < mk, a, zero))
                cand.append(tree_max(chunk))
            mk = tree_max(cand)
        for e in range(E):
            a = lbuf[e]
            w = jnp.where(a >= mk, a, zero)
            plsc.store_scatter(wbuf, [lane64 + (goff + e)], w)
        return carry

    lax.fori_loop(0, apw // LANES, group, 0)
    pltpu.sync_copy(wbuf, w_hbm.at[pl.ds(base, flat)])


def _out_kernel(xf_ref, w_ref, la_ref, lb_ref, wm_ref, limes_ref, o_ref):
    xb = xf_ref[...]
    xw = lax.dot_general(xb.astype(jnp.bfloat16), w_ref[...].astype(jnp.bfloat16),
                         (((1,), (1,)), ((), ())),
                         preferred_element_type=jnp.float32)
    delta = lax.dot_general(xb, la_ref[...], (((1,), (1,)), ((), ())),
                            preferred_element_type=jnp.float32)
    wm = wm_ref[...]
    s = jnp.maximum(jnp.sum(wm, axis=-1, keepdims=True), 1e-9)
    mix = lax.dot_general(wm, limes_ref[...], (((1,), (0,)), ((), ())),
                          preferred_element_type=jnp.float32)
    p_mix = mix / s
    hbc = TB // NGRAM
    rows = lax.broadcasted_iota(jnp.int32, (TB, hbc), 0) // NGRAM
    cols = lax.broadcasted_iota(jnp.int32, (TB, hbc), 1)
    erep = (rows == cols).astype(jnp.float32)
    p_full = lax.dot_general(erep, p_mix, (((1,), (0,)), ((), ())),
                             preferred_element_type=jnp.float32)
    q = delta * p_full
    lora = lax.dot_general(q, lb_ref[...], (((1,), (1,)), ((), ())),
                           preferred_element_type=jnp.float32)
    o_ref[...] = xw + lora * (ALPHA / R)


def kernel(x, weight, lora_A, lora_B, router_h, router_d, limes):
    Bsz, T, _ = x.shape
    na = (T // NGRAM) * Bsz
    bt = Bsz * T
    x3 = x.reshape(na, NGRAM * IN_F)
    xf = x.reshape(bt, IN_F)

    h_all, ds_all, hs, dss = pl.pallas_call(
        _anchor_kernel,
        grid=(na // AB,),
        in_specs=[
            pl.BlockSpec((AB, IN_F), lambda j: (j, NGRAM - 1)),
            pl.BlockSpec((R, IN_F), lambda j: (0, 0)),
            pl.BlockSpec((IN_F, E), lambda j: (0, 0)),
            pl.BlockSpec((R, E), lambda j: (0, 0)),
        ],
        out_specs=[
            pl.BlockSpec((AB, E), lambda j: (j, 0)),
            pl.BlockSpec((AB, E), lambda j: (j, 0)),
            pl.BlockSpec((1, 1), lambda j: (0, 0)),
            pl.BlockSpec((1, 1), lambda j: (0, 0)),
        ],
        out_shape=[
            jax.ShapeDtypeStruct((na, E), jnp.float32),
            jax.ShapeDtypeStruct((na, E), jnp.float32),
            jax.ShapeDtypeStruct((1, 1), jnp.float32),
            jax.ShapeDtypeStruct((1, 1), jnp.float32),
        ],
    )(x3, lora_A, router_h, router_d)

    eps = 1e-6
    c1 = (1.0 - GAMMA) / (jnp.maximum(hs[0, 0], eps) * max(TEMP, eps))
    c2 = GAMMA / (jnp.maximum(dss[0, 0], eps) * max(TEMP, eps))
    cvec = jnp.concatenate([jnp.full((LANES,), 1.0) * c1,
                            jnp.full((LANES,), 1.0) * c2]).astype(jnp.float32)

    apw = na // NW
    sc_route = functools.partial(
        pl.kernel,
        out_type=jax.ShapeDtypeStruct((na * E,), jnp.float32),
        mesh=plsc.VectorSubcoreMesh(core_axis_name="c", subcore_axis_name="s"),
        compiler_params=pltpu.CompilerParams(needs_layout_passes=False),
        scratch_types=[
            pltpu.VMEM((apw * E,), jnp.float32),
            pltpu.VMEM((apw * E,), jnp.float32),
            pltpu.VMEM((apw * E,), jnp.float32),
            pltpu.VMEM((2 * LANES,), jnp.float32),
            pltpu.VMEM((E, LANES), jnp.float32),
        ],
    )(_sc_routing)
    w_all = sc_route(h_all.reshape(-1), ds_all.reshape(-1), cvec).reshape(na, E)

    out = pl.pallas_call(
        _out_kernel,
        grid=(bt // TB,),
        in_specs=[
            pl.BlockSpec((TB, IN_F), lambda j: (j, 0)),
            pl.BlockSpec((OUT_F, IN_F), lambda j: (0, 0)),
            pl.BlockSpec((R, IN_F), lambda j: (0, 0)),
            pl.BlockSpec((OUT_F, R), lambda j: (0, 0)),
            pl.BlockSpec((TB // NGRAM, E), lambda j: (j, 0)),
            pl.BlockSpec((E, R), lambda j: (0, 0)),
        ],
        out_specs=pl.BlockSpec((TB, OUT_F), lambda j: (j, 0)),
        out_shape=jax.ShapeDtypeStruct((bt, OUT_F), jnp.float32),
    )(xf, weight, lora_A, lora_B, w_all, limes)

    return out.reshape(Bsz, T, OUT_F)


# speedup vs baseline: 1.1405x; 1.0520x over previous
import functools

import jax
import jax.numpy as jnp
from jax import lax
from jax.experimental import pallas as pl
from jax.experimental.pallas import tpu as pltpu
from jax.experimental.pallas import tpu_sc as plsc

IN_F = 768
OUT_F = 768
R = 8
E = 64
TOPK = 8
NGRAM = 2
GAMMA = 0.5
TEMP = 1.0
ALPHA = 16.0

AB = 1024
TB = 1024

NW = 32
NC = 2
LANES = 16


def _anchor_kernel(x3_ref, la_ref, rh_ref, rd_ref, h_ref, ds_ref, hs_ref, dss_ref):
    j = pl.program_id(0)
    xa = x3_ref[...]
    h = lax.dot_general(xa, rh_ref[...], (((1,), (0,)), ((), ())),
                        preferred_element_type=jnp.float32)
    da = lax.dot_general(xa, la_ref[...], (((1,), (1,)), ((), ())),
                         preferred_element_type=jnp.float32)
    ds = lax.dot_general(da, rd_ref[...], (((1,), (0,)), ((), ())),
                         preferred_element_type=jnp.float32)
    h_ref[...] = h
    ds_ref[...] = ds
    hmax = jnp.max(jnp.abs(h), keepdims=True).reshape(1, 1)
    dmax = jnp.max(jnp.abs(ds), keepdims=True).reshape(1, 1)

    @pl.when(j == 0)
    def _():
        hs_ref[...] = hmax
        dss_ref[...] = dmax

    @pl.when(j > 0)
    def _():
        hs_ref[...] = jnp.maximum(hs_ref[...], hmax)
        dss_ref[...] = jnp.maximum(dss_ref[...], dmax)


def _zprep_kernel(h_ref, ds_ref, hs_ref, dss_ref, zt_ref):
    eps = 1e-6
    c1 = (1.0 - GAMMA) / (jnp.maximum(hs_ref[...], eps) * max(TEMP, eps))
    c2 = GAMMA / (jnp.maximum(dss_ref[...], eps) * max(TEMP, eps))
    logits = c1 * h_ref[...] + c2 * ds_ref[...]
    m = jnp.max(logits, axis=-1, keepdims=True)
    z = jnp.exp(logits - m)
    zt_ref[...] = z


def _sc_topk(z_hbm, w_hbm, zbuf, wbuf, tbuf):
    wid = lax.axis_index("s") * NC + lax.axis_index("c")
    apw = (2 * 8192 // NGRAM) // NW
    flat = apw * E
    base = wid * flat
    pltpu.sync_copy(z_hbm.at[pl.ds(base, flat)], zbuf)

    def tree_max(vals):
        while len(vals) > 1:
            vals = [jnp.maximum(vals[2 * i], vals[2 * i + 1])
                    for i in range(len(vals) // 2)] + vals[len(vals) & ~1:]
        return vals[0]

    zero = jnp.full((LANES,), 0.0, dtype=jnp.float32)
    lane64 = lax.broadcasted_iota(jnp.int32, (LANES,), 0) * E

    def group(g, carry):
        goff = g * (LANES * E)
        for e in range(E):
            tbuf[e] = plsc.load_gather(zbuf, [lane64 + (goff + e)])
        mk = jnp.full((LANES,), 1.0, dtype=jnp.float32)
        for _ in range(TOPK - 1):
            cand = []
            for c in range(4):
                chunk = []
                for i in range(LANES):
                    a = tbuf[c * LANES + i]
                    chunk.append(jnp.where(a < mk, a, zero))
                cand.append(tree_max(chunk))
            mk = tree_max(cand)
        for e in range(E):
            a = tbuf[e]
            plsc.store_scatter(wbuf, [lane64 + (goff + e)],
                               jnp.where(a >= mk, a, zero))
        return carry

    lax.fori_loop(0, apw // LANES, group, 0)
    pltpu.sync_copy(wbuf, w_hbm.at[pl.ds(base, flat)])


def _out_kernel(xf_ref, w_ref, la_ref, lb_ref, wm_ref, limes_ref, o_ref):
    xb = xf_ref[...]
    xw = lax.dot_general(xb.astype(jnp.bfloat16), w_ref[...].astype(jnp.bfloat16),
                         (((1,), (1,)), ((), ())),
                         preferred_element_type=jnp.float32)
    delta = lax.dot_general(xb, la_ref[...], (((1,), (1,)), ((), ())),
                            preferred_element_type=jnp.float32)
    wm = wm_ref[...]
    s = jnp.maximum(jnp.sum(wm, axis=-1, keepdims=True), 1e-9)
    mix = lax.dot_general(wm, limes_ref[...], (((1,), (0,)), ((), ())),
                          preferred_element_type=jnp.float32)
    p_mix = mix / s
    hbc = TB // NGRAM
    rows = lax.broadcasted_iota(jnp.int32, (TB, hbc), 0) // NGRAM
    cols = lax.broadcasted_iota(jnp.int32, (TB, hbc), 1)
    erep = (rows == cols).astype(jnp.float32)
    p_full = lax.dot_general(erep, p_mix, (((1,), (0,)), ((), ())),
                             preferred_element_type=jnp.float32)
    q = delta * p_full
    lora = lax.dot_general(q, lb_ref[...], (((1,), (1,)), ((), ())),
                           preferred_element_type=jnp.float32)
    o_ref[...] = xw + lora * (ALPHA / R)


def kernel(x, weight, lora_A, lora_B, router_h, router_d, limes):
    Bsz, T, _ = x.shape
    na = (T // NGRAM) * Bsz
    bt = Bsz * T
    x3 = x.reshape(na, NGRAM * IN_F)
    xf = x.reshape(bt, IN_F)

    h_all, ds_all, hs, dss = pl.pallas_call(
        _anchor_kernel,
        grid=(na // AB,),
        in_specs=[
            pl.BlockSpec((AB, IN_F), lambda j: (j, NGRAM - 1)),
            pl.BlockSpec((R, IN_F), lambda j: (0, 0)),
            pl.BlockSpec((IN_F, E), lambda j: (0, 0)),
            pl.BlockSpec((R, E), lambda j: (0, 0)),
        ],
        out_specs=[
            pl.BlockSpec((AB, E), lambda j: (j, 0)),
            pl.BlockSpec((AB, E), lambda j: (j, 0)),
            pl.BlockSpec((1, 1), lambda j: (0, 0)),
            pl.BlockSpec((1, 1), lambda j: (0, 0)),
        ],
        out_shape=[
            jax.ShapeDtypeStruct((na, E), jnp.float32),
            jax.ShapeDtypeStruct((na, E), jnp.float32),
            jax.ShapeDtypeStruct((1, 1), jnp.float32),
            jax.ShapeDtypeStruct((1, 1), jnp.float32),
        ],
    )(x3, lora_A, router_h, router_d)

    z_t = pl.pallas_call(
        _zprep_kernel,
        grid=(na // AB,),
        in_specs=[
            pl.BlockSpec((AB, E), lambda j: (j, 0)),
            pl.BlockSpec((AB, E), lambda j: (j, 0)),
            pl.BlockSpec((1, 1), lambda j: (0, 0)),
            pl.BlockSpec((1, 1), lambda j: (0, 0)),
        ],
        out_specs=pl.BlockSpec((AB, E), lambda j: (j, 0)),
        out_shape=jax.ShapeDtypeStruct((na, E), jnp.float32),
    )(h_all, ds_all, hs, dss)

    apw = na // NW
    sc_topk = functools.partial(
        pl.kernel,
        out_type=jax.ShapeDtypeStruct((na * E,), jnp.float32),
        mesh=plsc.VectorSubcoreMesh(core_axis_name="c", subcore_axis_name="s"),
        compiler_params=pltpu.CompilerParams(needs_layout_passes=False),
        scratch_types=[
            pltpu.VMEM((apw * E,), jnp.float32),
            pltpu.VMEM((apw * E,), jnp.float32),
            pltpu.VMEM((E, LANES), jnp.float32),
        ],
    )(_sc_topk)
    w_t = sc_topk(z_t.reshape(-1)).reshape(na, E)

    out = pl.pallas_call(
        _out_kernel,
        grid=(bt // TB,),
        in_specs=[
            pl.BlockSpec((TB, IN_F), lambda j: (j, 0)),
            pl.BlockSpec((OUT_F, IN_F), lambda j: (0, 0)),
            pl.BlockSpec((R, IN_F), lambda j: (0, 0)),
            pl.BlockSpec((OUT_F, R), lambda j: (0, 0)),
            pl.BlockSpec((TB // NGRAM, E), lambda j: (j, 0)),
            pl.BlockSpec((E, R), lambda j: (0, 0)),
        ],
        out_specs=pl.BlockSpec((TB, OUT_F), lambda j: (j, 0)),
        out_shape=jax.ShapeDtypeStruct((bt, OUT_F), jnp.float32),
    )(xf, weight, lora_A, lora_B, w_t, limes)

    return out.reshape(Bsz, T, OUT_F)


# speedup vs baseline: 1.1801x; 1.0347x over previous
import functools

import jax
import jax.numpy as jnp
from jax import lax
from jax.experimental import pallas as pl
from jax.experimental.pallas import tpu as pltpu
from jax.experimental.pallas import tpu_sc as plsc

IN_F = 768
OUT_F = 768
R = 8
E = 64
TOPK = 8
NGRAM = 2
GAMMA = 0.5
TEMP = 1.0
ALPHA = 16.0

AB = 1024
TB = 1024

NW = 32
NC = 2
LANES = 16


def _anchor_z_kernel(x3_ref, la_ref, rh_ref, rd_ref, z_ref, h_scr, ds_scr, s_scr):
    p = pl.program_id(0)
    j = pl.program_id(1)
    sl = pl.ds(j * AB, AB)

    @pl.when(p == 0)
    def _():
        xa = x3_ref[...]
        h = lax.dot_general(xa, rh_ref[...], (((1,), (0,)), ((), ())),
                            preferred_element_type=jnp.float32)
        da = lax.dot_general(xa, la_ref[...], (((1,), (1,)), ((), ())),
                             preferred_element_type=jnp.float32)
        ds = lax.dot_general(da, rd_ref[...], (((1,), (0,)), ((), ())),
                             preferred_element_type=jnp.float32)
        h_scr[sl, :] = h
        ds_scr[sl, :] = ds
        hmax = jnp.max(jnp.abs(h))
        dmax = jnp.max(jnp.abs(ds))

        @pl.when(j == 0)
        def _():
            s_scr[0] = hmax
            s_scr[1] = dmax

        @pl.when(j > 0)
        def _():
            s_scr[0] = jnp.maximum(s_scr[0], hmax)
            s_scr[1] = jnp.maximum(s_scr[1], dmax)

    @pl.when(p == 1)
    def _():
        eps = 1e-6
        c1 = (1.0 - GAMMA) / (jnp.maximum(s_scr[0], eps) * max(TEMP, eps))
        c2 = GAMMA / (jnp.maximum(s_scr[1], eps) * max(TEMP, eps))
        logits = c1 * h_scr[sl, :] + c2 * ds_scr[sl, :]
        m = jnp.max(logits, axis=-1, keepdims=True)
        z_ref[...] = jnp.exp(logits - m)


def _sc_topk(z_hbm, w_hbm, zbuf, wbuf, tbuf):
    wid = lax.axis_index("s") * NC + lax.axis_index("c")
    apw = (2 * 8192 // NGRAM) // NW
    flat = apw * E
    base = wid * flat
    pltpu.sync_copy(z_hbm.at[pl.ds(base, flat)], zbuf)

    def tree_max(vals):
        while len(vals) > 1:
            vals = [jnp.maximum(vals[2 * i], vals[2 * i + 1])
                    for i in range(len(vals) // 2)] + vals[len(vals) & ~1:]
        return vals[0]

    zero = jnp.full((LANES,), 0.0, dtype=jnp.float32)
    lane64 = lax.broadcasted_iota(jnp.int32, (LANES,), 0) * E

    def group(g, carry):
        goff = g * (LANES * E)
        for e in range(E):
            tbuf[e] = plsc.load_gather(zbuf, [lane64 + (goff + e)])
        mk = jnp.full((LANES,), 1.0, dtype=jnp.float32)
        for _ in range(TOPK - 1):
            cand = []
            for c in range(4):
                chunk = []
                for i in range(LANES):
                    a = tbuf[c * LANES + i]
                    chunk.append(jnp.where(a < mk, a, zero))
                cand.append(tree_max(chunk))
            mk = tree_max(cand)
        for e in range(E):
            a = tbuf[e]
            plsc.store_scatter(wbuf, [lane64 + (goff + e)],
                               jnp.where(a >= mk, a, zero))
        return carry

    lax.fori_loop(0, apw // LANES, group, 0)
    pltpu.sync_copy(wbuf, w_hbm.at[pl.ds(base, flat)])


def _out_kernel(xf_ref, w_ref, la_ref, lb_ref, wm_ref, limes_ref, o_ref):
    xb = xf_ref[...]
    xw = lax.dot_general(xb.astype(jnp.bfloat16), w_ref[...].astype(jnp.bfloat16),
                         (((1,), (1,)), ((), ())),
                         preferred_element_type=jnp.float32)
    delta = lax.dot_general(xb, la_ref[...], (((1,), (1,)), ((), ())),
                            preferred_element_type=jnp.float32)
    wm = wm_ref[...]
    s = jnp.maximum(jnp.sum(wm, axis=-1, keepdims=True), 1e-9)
    mix = lax.dot_general(wm, limes_ref[...], (((1,), (0,)), ((), ())),
                          preferred_element_type=jnp.float32)
    p_mix = mix / s
    hbc = TB // NGRAM
    rows = lax.broadcasted_iota(jnp.int32, (TB, hbc), 0) // NGRAM
    cols = lax.broadcasted_iota(jnp.int32, (TB, hbc), 1)
    erep = (rows == cols).astype(jnp.float32)
    p_full = lax.dot_general(erep, p_mix, (((1,), (0,)), ((), ())),
                             preferred_element_type=jnp.float32)
    q = delta * p_full
    lora = lax.dot_general(q, lb_ref[...], (((1,), (1,)), ((), ())),
                           preferred_element_type=jnp.float32)
    o_ref[...] = xw + lora * (ALPHA / R)


def kernel(x, weight, lora_A, lora_B, router_h, router_d, limes):
    Bsz, T, _ = x.shape
    na = (T // NGRAM) * Bsz
    bt = Bsz * T
    x3 = x.reshape(na, NGRAM * IN_F)
    xf = x.reshape(bt, IN_F)

    nb = na // AB
    z_t = pl.pallas_call(
        _anchor_z_kernel,
        grid=(2, nb),
        in_specs=[
            pl.BlockSpec((AB, IN_F),
                         lambda p, j: (jnp.where(p == 0, j, nb - 1), NGRAM - 1)),
            pl.BlockSpec((R, IN_F), lambda p, j: (0, 0)),
            pl.BlockSpec((IN_F, E), lambda p, j: (0, 0)),
            pl.BlockSpec((R, E), lambda p, j: (0, 0)),
        ],
        out_specs=pl.BlockSpec((AB, E), lambda p, j: (j, 0)),
        out_shape=jax.ShapeDtypeStruct((na, E), jnp.float32),
        scratch_shapes=[
            pltpu.VMEM((na, E), jnp.float32),
            pltpu.VMEM((na, E), jnp.float32),
            pltpu.SMEM((2,), jnp.float32),
        ],
    )(x3, lora_A, router_h, router_d)

    apw = na // NW
    sc_topk = functools.partial(
        pl.kernel,
        out_type=jax.ShapeDtypeStruct((na * E,), jnp.float32),
        mesh=plsc.VectorSubcoreMesh(core_axis_name="c", subcore_axis_name="s"),
        compiler_params=pltpu.CompilerParams(needs_layout_passes=False),
        scratch_types=[
            pltpu.VMEM((apw * E,), jnp.float32),
            pltpu.VMEM((apw * E,), jnp.float32),
            pltpu.VMEM((E, LANES), jnp.float32),
        ],
    )(_sc_topk)
    w_t = sc_topk(z_t.reshape(-1)).reshape(na, E)

    out = pl.pallas_call(
        _out_kernel,
        grid=(bt // TB,),
        in_specs=[
            pl.BlockSpec((TB, IN_F), lambda j: (j, 0)),
            pl.BlockSpec((OUT_F, IN_F), lambda j: (0, 0)),
            pl.BlockSpec((R, IN_F), lambda j: (0, 0)),
            pl.BlockSpec((OUT_F, R), lambda j: (0, 0)),
            pl.BlockSpec((TB // NGRAM, E), lambda j: (j, 0)),
            pl.BlockSpec((E, R), lambda j: (0, 0)),
        ],
        out_specs=pl.BlockSpec((TB, OUT_F), lambda j: (j, 0)),
        out_shape=jax.ShapeDtypeStruct((bt, OUT_F), jnp.float32),
    )(xf, weight, lora_A, lora_B, w_t, limes)

    return out.reshape(Bsz, T, OUT_F)
